# baseline (speedup 1.0000x reference)
import functools
import math

import numpy as np
import jax
import jax.numpy as jnp
from jax import lax
from jax.experimental import pallas as pl
from jax.experimental.pallas import tpu as pltpu

_NH = 8
_F32 = jnp.float32


def _ln(x, g, b, eps=1e-5):
    mu = jnp.mean(x, axis=-1, keepdims=True)
    var = jnp.mean((x - mu) ** 2, axis=-1, keepdims=True)
    return (x - mu) * lax.rsqrt(var + eps) * g + b


def _pos_emb_np(L, D):
    half = D // 2
    freqs = np.exp(np.arange(half) * -(math.log(10000.0) / (half - 1)))
    pos = np.arange(1, L + 1, dtype=np.float32)
    args = pos[:, None] * freqs[None, :]
    pe = np.concatenate([np.sin(args), np.cos(args)], axis=1)
    if D % 2 == 1:
        pe = np.concatenate([pe, np.zeros((L, 1))], axis=1)
    return pe.astype(np.float32)


def _layer_body(x_scr, kv, bias, p, *, cross):
    ln0g, ln0b, wq, bq, wkv, bkv, wo, bo, ln1g, ln1b, w1, b1, w2, b2 = p
    x = x_scr[...]
    D = x.shape[-1]
    dh = D // _NH
    g0 = ln0g[0]
    b0 = ln0b[0]
    xn = _ln(x, g0, b0)
    kvn = _ln(kv, g0, b0) if cross else xn

    q = jnp.dot(xn, wq[0], preferred_element_type=_F32) + bq[0]
    kvp = jnp.dot(kvn, wkv[0], preferred_element_type=_F32) + bkv[0]
    k = kvp[:, :D]
    v = kvp[:, D:]

    heads = []
    for h in range(_NH):
        qh = q[:, h * dh:(h + 1) * dh]
        kh = k[:, h * dh:(h + 1) * dh]
        vh = v[:, h * dh:(h + 1) * dh]
        s = lax.dot_general(qh, kh, (((1,), (1,)), ((), ())),
                            preferred_element_type=_F32)
        s = s + bias
        m = jnp.max(s, axis=-1, keepdims=True)
        e = jnp.exp(s - m)
        e = e / jnp.sum(e, axis=-1, keepdims=True)
        heads.append(jnp.dot(e, vh, preferred_element_type=_F32))
    attn = jnp.concatenate(heads, axis=-1)
    x = x + jnp.dot(attn, wo[0], preferred_element_type=_F32) + bo[0]

    xn2 = _ln(x, ln1g[0], ln1b[0])
    h1 = jnp.maximum(jnp.dot(xn2, w1[0], preferred_element_type=_F32) + b1[0],
                     0.0)
    x = x + jnp.dot(h1, w2[0], preferred_element_type=_F32) + b2[0]
    x_scr[...] = x
    return x


def _cross_kernel(qe_ref, qw_ref, qb_ref, ke_ref, kw_ref, kb_ref,
                  pos_ref, bias_ref, *rest, nl):
    p0 = rest[:14]
    p1 = rest[14:28]
    f1g, f1b = rest[28], rest[29]
    o_ref = rest[30]
    x_scr, kv_scr = rest[31], rest[32]

    b = pl.program_id(0)
    l = pl.program_id(1)

    @pl.when(l == 0)
    def _():
        hq = jnp.dot(qe_ref[...], qw_ref[...],
                     preferred_element_type=_F32) + qb_ref[...]
        hk = jnp.dot(ke_ref[...], kw_ref[...],
                     preferred_element_type=_F32) + kb_ref[...]
        pos = pos_ref[...]
        x_scr[...] = jnp.where(b == 0, hk, hq) + pos
        kv_scr[...] = jnp.where(b == 0, hq, hk) + pos

    def run(p):
        x = _layer_body(x_scr, kv_scr[...], bias_ref[...], p, cross=True)

        @pl.when(l == nl - 1)
        def _():
            o_ref[0] = _ln(x, f1g[0], f1b[0])

    @pl.when(b == 0)
    def _():
        run(p0)

    @pl.when(b == 1)
    def _():
        run(p1)


def _mem_kernel(x_ref, pos_ref, bias_ref, pool_ref, w1_ref, b1_ref,
                w2_ref, b2_ref, gT_ref, cb_ref, *rest, nl, scale):
    p0 = rest[:14]
    p1 = rest[14:28]
    f2g, f2b = rest[28], rest[29]
    o_ref = rest[30]
    x_scr, hk_scr = rest[31], rest[32]

    b = pl.program_id(0)
    l = pl.program_id(1)

    @pl.when(l == 0)
    def _():
        x_scr[...] = scale * x_ref[0] + pos_ref[...]

    @pl.when(b == 0)
    def _():
        x = _layer_body(x_scr, None, bias_ref[...], p0, cross=False)

        @pl.when(l == nl - 1)
        def _():
            hk_scr[...] = _ln(x, f2g[0], f2b[0])

    @pl.when(b == 1)
    def _():
        x = _layer_body(x_scr, None, bias_ref[...], p1, cross=False)

        @pl.when(l == nl - 1)
        def _():
            h_qs = _ln(x, f2g[0], f2b[0])
            pool = pool_ref[...]
            ks = jnp.dot(pool, hk_scr[...], preferred_element_type=_F32)
            qs = jnp.dot(pool, h_qs, preferred_element_type=_F32)
            last = jnp.concatenate([ks, qs], axis=-1)
            h = jnp.maximum(
                jnp.dot(last, w1_ref[...], preferred_element_type=_F32)
                + b1_ref[...], 0.0)
            out = (jnp.dot(h, w2_ref[...], preferred_element_type=_F32)
                   + b2_ref[...])
            sim = (jnp.dot(out, gT_ref[...], preferred_element_type=_F32)
                   + cb_ref[...])
            m = jnp.max(sim, axis=-1, keepdims=True)
            lse = m + jnp.log(jnp.sum(jnp.exp(sim - m), axis=-1,
                                      keepdims=True))
            o_ref[...] = sim - lse


def _stack_specs(shapes, own_branch, nl):
    specs = []
    for shp in shapes:
        if own_branch == 0:
            def idx(b, l, _n=nl):
                return (jnp.where(b == 0, l, _n - 1), 0, 0)
        else:
            def idx(b, l):
                return (jnp.where(b == 1, l, 0), 0, 0)
        specs.append(pl.BlockSpec((1,) + shp[1:], idx))
    return specs


def kernel(he_ques, he_kg, emb, q2h_w, q2h_b, k2h_w, k2h_b,
           kq_ln0g, kq_ln0b, kq_wq, kq_bq, kq_wkv, kq_bkv, kq_wo, kq_bo,
           kq_ln1g, kq_ln1b, kq_w1, kq_b1, kq_w2, kq_b2, kq_fg, kq_fb,
           qk_ln0g, qk_ln0b, qk_wq, qk_bq, qk_wkv, qk_bkv, qk_wo, qk_bo,
           qk_ln1g, qk_ln1b, qk_w1, qk_b1, qk_w2, qk_b2, qk_fg, qk_fb,
           kmem_ln0g, kmem_ln0b, kmem_wq, kmem_bq, kmem_wkv, kmem_bkv,
           kmem_wo, kmem_bo, kmem_ln1g, kmem_ln1b, kmem_w1, kmem_b1,
           kmem_w2, kmem_b2, kmem_fg, kmem_fb,
           qmem_ln0g, qmem_ln0b, qmem_wq, qmem_bq, qmem_wkv, qmem_bkv,
           qmem_wo, qmem_bo, qmem_ln1g, qmem_ln1b, qmem_w1, qmem_b1,
           qmem_w2, qmem_b2, qmem_fg, qmem_fb,
           proj1_w, proj1_b, proj2_w, proj2_b, glove_T):
    B, Lq, _ = he_ques.shape
    _, Lk, _ = he_kg.shape
    D = q2h_w.shape[1]
    R = B * Lq
    assert Lq == Lk, "fused dual-branch layout needs equal slab shapes"
    scale = math.sqrt(D)

    q_emb = emb[he_ques].reshape(R, -1)
    k_emb = emb[he_kg].reshape(R, -1)

    pos = jnp.asarray(np.tile(_pos_emb_np(Lq, D), (B, 1)))
    blk = np.repeat(np.arange(B), Lq)
    bias = jnp.asarray(
        np.where(blk[:, None] == blk[None, :], 0.0, -1e30).astype(np.float32))
    pool = jnp.asarray(
        (blk[None, :] == np.arange(B)[:, None]).astype(np.float32))

    qw = q2h_w * scale
    qb = q2h_b * scale
    kw = k2h_w * scale
    kb = k2h_b * scale

    kq = (kq_ln0g, kq_ln0b, kq_wq, kq_bq, kq_wkv, kq_bkv, kq_wo, kq_bo,
          kq_ln1g, kq_ln1b, kq_w1, kq_b1, kq_w2, kq_b2)
    qk = (qk_ln0g, qk_ln0b, qk_wq, qk_bq, qk_wkv, qk_bkv, qk_wo, qk_bo,
          qk_ln1g, qk_ln1b, qk_w1, qk_b1, qk_w2, qk_b2)
    kmem = (kmem_ln0g, kmem_ln0b, kmem_wq, kmem_bq, kmem_wkv, kmem_bkv,
            kmem_wo, kmem_bo, kmem_ln1g, kmem_ln1b, kmem_w1, kmem_b1,
            kmem_w2, kmem_b2)
    qmem = (qmem_ln0g, qmem_ln0b, qmem_wq, qmem_bq, qmem_wkv, qmem_bkv,
            qmem_wo, qmem_bo, qmem_ln1g, qmem_ln1b, qmem_w1, qmem_b1,
            qmem_w2, qmem_b2)

    f1g = jnp.stack([kq_fg, qk_fg])
    f1b = jnp.stack([kq_fb, qk_fb])
    f2g = jnp.stack([kmem_fg, qmem_fg])
    f2b = jnp.stack([kmem_fb, qmem_fb])

    sem = pltpu.CompilerParams(
        dimension_semantics=("arbitrary", "arbitrary"),
        vmem_limit_bytes=64 * 1024 * 1024)
    res = lambda shp: pl.BlockSpec(shp, lambda b, l: (0,) * len(shp))
    fspec = pl.BlockSpec((1, 1, D), lambda b, l: (b, 0, 0))

    nl1 = kq_wq.shape[0]
    in_specs = [
        res(q_emb.shape), res(qw.shape), res(qb.shape),
        res(k_emb.shape), res(kw.shape), res(kb.shape),
        res((R, D)), res((R, R)),
    ]
    in_specs += _stack_specs([p.shape for p in kq], 0, nl1)
    in_specs += _stack_specs([p.shape for p in qk], 1, nl1)
    in_specs += [fspec, fspec]
    h1 = pl.pallas_call(
        functools.partial(_cross_kernel, nl=nl1),
        out_shape=jax.ShapeDtypeStruct((2, R, D), jnp.float32),
        grid=(2, nl1),
        in_specs=in_specs,
        out_specs=pl.BlockSpec((1, R, D), lambda b, l: (b, 0, 0)),
        scratch_shapes=[pltpu.VMEM((R, D), jnp.float32),
                        pltpu.VMEM((R, D), jnp.float32)],
        compiler_params=sem,
    )(q_emb, qw, qb, k_emb, kw, kb, pos, bias, *kq, *qk, f1g, f1b)

    NO = proj2_w.shape[1]
    C = glove_T.shape[1]
    CP = ((C + 127) // 128) * 128
    gT_pad = jnp.pad(glove_T, ((0, 0), (0, CP - C)))
    cand_bias = jnp.concatenate(
        [jnp.zeros((1, C), jnp.float32),
         jnp.full((1, CP - C), -1e30, jnp.float32)], axis=1)

    nl2 = kmem_wq.shape[0]
    in_specs = [
        pl.BlockSpec((1, R, D), lambda b, l: (b, 0, 0)),
        res((R, D)), res((R, R)), res((B, R)),
        res(proj1_w.shape), res(proj1_b.shape),
        res(proj2_w.shape), res(proj2_b.shape),
        res((NO, CP)), res((1, CP)),
    ]
    in_specs += _stack_specs([p.shape for p in kmem], 0, nl2)
    in_specs += _stack_specs([p.shape for p in qmem], 1, nl2)
    in_specs += [fspec, fspec]
    pred = pl.pallas_call(
        functools.partial(_mem_kernel, nl=nl2, scale=scale),
        out_shape=jax.ShapeDtypeStruct((B, CP), jnp.float32),
        grid=(2, nl2),
        in_specs=in_specs,
        out_specs=pl.BlockSpec((B, CP), lambda b, l: (0, 0)),
        scratch_shapes=[pltpu.VMEM((R, D), jnp.float32),
                        pltpu.VMEM((R, D), jnp.float32)],
        compiler_params=sem,
    )(h1, pos, bias, pool, proj1_w, proj1_b, proj2_w, proj2_b,
      gT_pad, cand_bias, *kmem, *qmem, f2g, f2b)
    return pred[:, :C]


# speedup vs baseline: 1.1559x; 1.1559x over previous
import functools
import math

import numpy as np
import jax
import jax.numpy as jnp
from jax import lax
from jax.experimental import pallas as pl
from jax.experimental.pallas import tpu as pltpu

_NH = 8
_F32 = jnp.float32


def _ln(x, g, b, eps=1e-5):
    mu = jnp.mean(x, axis=-1, keepdims=True)
    var = jnp.mean((x - mu) ** 2, axis=-1, keepdims=True)
    return (x - mu) * lax.rsqrt(var + eps) * g + b


def _pos_emb_np(L, D):
    half = D // 2
    freqs = np.exp(np.arange(half) * -(math.log(10000.0) / (half - 1)))
    pos = np.arange(1, L + 1, dtype=np.float32)
    args = pos[:, None] * freqs[None, :]
    pe = np.concatenate([np.sin(args), np.cos(args)], axis=1)
    if D % 2 == 1:
        pe = np.concatenate([pe, np.zeros((L, 1))], axis=1)
    return pe.astype(np.float32)


def _layer_body(x_scr, kv, bias, p, *, cross):
    ln0g, ln0b, wq, bq, wkv, bkv, wo, bo, ln1g, ln1b, w1, b1, w2, b2 = p
    x = x_scr[...]
    D = x.shape[-1]
    dh = D // _NH
    g0 = ln0g[0]
    b0 = ln0b[0]
    xn = _ln(x, g0, b0)
    kvn = _ln(kv, g0, b0) if cross else xn

    q = jnp.dot(xn, wq[0], preferred_element_type=_F32) + bq[0]
    kvp = jnp.dot(kvn, wkv[0], preferred_element_type=_F32) + bkv[0]
    k = kvp[:, :D]
    v = kvp[:, D:]

    R = x.shape[0]
    T = bias.shape[0]
    tiles = [(h, g) for h in range(_NH) for g in range(R // T)]
    ss = {}
    for h, g in tiles:
        sl = slice(g * T, (g + 1) * T)
        cols = slice(h * dh, (h + 1) * dh)
        ss[h, g] = lax.dot_general(q[sl, cols], k[sl, cols],
                                   (((1,), (1,)), ((), ())),
                                   preferred_element_type=_F32)
    es = {}
    for h, g in tiles:
        s = ss[h, g] + bias
        m = jnp.max(s, axis=-1, keepdims=True)
        e = jnp.exp(s - m)
        es[h, g] = (e, 1.0 / jnp.sum(e, axis=-1, keepdims=True))
    heads = []
    for h in range(_NH):
        outs = []
        for g in range(R // T):
            sl = slice(g * T, (g + 1) * T)
            e, r = es[h, g]
            outs.append(
                jnp.dot(e, v[sl, h * dh:(h + 1) * dh],
                        preferred_element_type=_F32) * r)
        heads.append(jnp.concatenate(outs, axis=0))
    attn = jnp.concatenate(heads, axis=-1)
    x = x + jnp.dot(attn, wo[0], preferred_element_type=_F32) + bo[0]

    xn2 = _ln(x, ln1g[0], ln1b[0])
    h1 = jnp.maximum(jnp.dot(xn2, w1[0], preferred_element_type=_F32) + b1[0],
                     0.0)
    x = x + jnp.dot(h1, w2[0], preferred_element_type=_F32) + b2[0]
    x_scr[...] = x
    return x


def _cross_kernel(qe_ref, qw_ref, qb_ref, ke_ref, kw_ref, kb_ref,
                  pos_ref, bias_ref, *rest, nl):
    p0 = rest[:14]
    p1 = rest[14:28]
    f1g, f1b = rest[28], rest[29]
    o_ref = rest[30]
    x_scr, kv_scr = rest[31], rest[32]

    b = pl.program_id(0)
    l = pl.program_id(1)

    @pl.when(l == 0)
    def _():
        hq = jnp.dot(qe_ref[...], qw_ref[...],
                     preferred_element_type=_F32) + qb_ref[...]
        hk = jnp.dot(ke_ref[...], kw_ref[...],
                     preferred_element_type=_F32) + kb_ref[...]
        pos = pos_ref[...]
        x_scr[...] = jnp.where(b == 0, hk, hq) + pos
        kv_scr[...] = jnp.where(b == 0, hq, hk) + pos

    def run(p):
        x = _layer_body(x_scr, kv_scr[...], bias_ref[...], p, cross=True)

        @pl.when(l == nl - 1)
        def _():
            o_ref[0] = _ln(x, f1g[0], f1b[0])

    @pl.when(b == 0)
    def _():
        run(p0)

    @pl.when(b == 1)
    def _():
        run(p1)


def _mem_kernel(x_ref, pos_ref, bias_ref, pool_ref, w1_ref, b1_ref,
                w2_ref, b2_ref, gT_ref, cb_ref, *rest, nl, scale):
    p0 = rest[:14]
    p1 = rest[14:28]
    f2g, f2b = rest[28], rest[29]
    o_ref = rest[30]
    x_scr, hk_scr = rest[31], rest[32]

    b = pl.program_id(0)
    l = pl.program_id(1)

    @pl.when(l == 0)
    def _():
        x_scr[...] = scale * x_ref[0] + pos_ref[...]

    @pl.when(b == 0)
    def _():
        x = _layer_body(x_scr, None, bias_ref[...], p0, cross=False)

        @pl.when(l == nl - 1)
        def _():
            hk_scr[...] = _ln(x, f2g[0], f2b[0])

    @pl.when(b == 1)
    def _():
        x = _layer_body(x_scr, None, bias_ref[...], p1, cross=False)

        @pl.when(l == nl - 1)
        def _():
            h_qs = _ln(x, f2g[0], f2b[0])
            pool = pool_ref[...]
            ks = jnp.dot(pool, hk_scr[...], preferred_element_type=_F32)
            qs = jnp.dot(pool, h_qs, preferred_element_type=_F32)
            last = jnp.concatenate([ks, qs], axis=-1)
            h = jnp.maximum(
                jnp.dot(last, w1_ref[...], preferred_element_type=_F32)
                + b1_ref[...], 0.0)
            out = (jnp.dot(h, w2_ref[...], preferred_element_type=_F32)
                   + b2_ref[...])
            sim = (jnp.dot(out, gT_ref[...], preferred_element_type=_F32)
                   + cb_ref[...])
            m = jnp.max(sim, axis=-1, keepdims=True)
            lse = m + jnp.log(jnp.sum(jnp.exp(sim - m), axis=-1,
                                      keepdims=True))
            o_ref[...] = sim - lse


def _stack_specs(shapes, own_branch, nl):
    specs = []
    for shp in shapes:
        if own_branch == 0:
            def idx(b, l, _n=nl):
                return (jnp.where(b == 0, l, _n - 1), 0, 0)
        else:
            def idx(b, l):
                return (jnp.where(b == 1, l, 0), 0, 0)
        specs.append(pl.BlockSpec((1,) + shp[1:], idx))
    return specs


def kernel(he_ques, he_kg, emb, q2h_w, q2h_b, k2h_w, k2h_b,
           kq_ln0g, kq_ln0b, kq_wq, kq_bq, kq_wkv, kq_bkv, kq_wo, kq_bo,
           kq_ln1g, kq_ln1b, kq_w1, kq_b1, kq_w2, kq_b2, kq_fg, kq_fb,
           qk_ln0g, qk_ln0b, qk_wq, qk_bq, qk_wkv, qk_bkv, qk_wo, qk_bo,
           qk_ln1g, qk_ln1b, qk_w1, qk_b1, qk_w2, qk_b2, qk_fg, qk_fb,
           kmem_ln0g, kmem_ln0b, kmem_wq, kmem_bq, kmem_wkv, kmem_bkv,
           kmem_wo, kmem_bo, kmem_ln1g, kmem_ln1b, kmem_w1, kmem_b1,
           kmem_w2, kmem_b2, kmem_fg, kmem_fb,
           qmem_ln0g, qmem_ln0b, qmem_wq, qmem_bq, qmem_wkv, qmem_bkv,
           qmem_wo, qmem_bo, qmem_ln1g, qmem_ln1b, qmem_w1, qmem_b1,
           qmem_w2, qmem_b2, qmem_fg, qmem_fb,
           proj1_w, proj1_b, proj2_w, proj2_b, glove_T):
    B, Lq, _ = he_ques.shape
    _, Lk, _ = he_kg.shape
    D = q2h_w.shape[1]
    R = B * Lq
    assert Lq == Lk, "fused dual-branch layout needs equal slab shapes"
    scale = math.sqrt(D)

    q_emb = emb[he_ques].reshape(R, -1)
    k_emb = emb[he_kg].reshape(R, -1)

    pos = jnp.asarray(np.tile(_pos_emb_np(Lq, D), (B, 1)))
    blk = np.repeat(np.arange(B), Lq)
    pool = jnp.asarray(
        (blk[None, :] == np.arange(B)[:, None]).astype(np.float32))
    T = math.gcd(R, 128)
    if T % Lq != 0:
        T = R
    tb = np.repeat(np.arange(T // Lq), Lq)
    bias = jnp.asarray(
        np.where(tb[:, None] == tb[None, :], 0.0, -1e30).astype(np.float32))

    qw = q2h_w * scale
    qb = q2h_b * scale
    kw = k2h_w * scale
    kb = k2h_b * scale

    kq = (kq_ln0g, kq_ln0b, kq_wq, kq_bq, kq_wkv, kq_bkv, kq_wo, kq_bo,
          kq_ln1g, kq_ln1b, kq_w1, kq_b1, kq_w2, kq_b2)
    qk = (qk_ln0g, qk_ln0b, qk_wq, qk_bq, qk_wkv, qk_bkv, qk_wo, qk_bo,
          qk_ln1g, qk_ln1b, qk_w1, qk_b1, qk_w2, qk_b2)
    kmem = (kmem_ln0g, kmem_ln0b, kmem_wq, kmem_bq, kmem_wkv, kmem_bkv,
            kmem_wo, kmem_bo, kmem_ln1g, kmem_ln1b, kmem_w1, kmem_b1,
            kmem_w2, kmem_b2)
    qmem = (qmem_ln0g, qmem_ln0b, qmem_wq, qmem_bq, qmem_wkv, qmem_bkv,
            qmem_wo, qmem_bo, qmem_ln1g, qmem_ln1b, qmem_w1, qmem_b1,
            qmem_w2, qmem_b2)

    f1g = jnp.stack([kq_fg, qk_fg])
    f1b = jnp.stack([kq_fb, qk_fb])
    f2g = jnp.stack([kmem_fg, qmem_fg])
    f2b = jnp.stack([kmem_fb, qmem_fb])

    sem = pltpu.CompilerParams(
        dimension_semantics=("arbitrary", "arbitrary"),
        vmem_limit_bytes=64 * 1024 * 1024)
    res = lambda shp: pl.BlockSpec(shp, lambda b, l: (0,) * len(shp))
    fspec = pl.BlockSpec((1, 1, D), lambda b, l: (b, 0, 0))

    nl1 = kq_wq.shape[0]
    in_specs = [
        res(q_emb.shape), res(qw.shape), res(qb.shape),
        res(k_emb.shape), res(kw.shape), res(kb.shape),
        res((R, D)), res((T, T)),
    ]
    in_specs += _stack_specs([p.shape for p in kq], 0, nl1)
    in_specs += _stack_specs([p.shape for p in qk], 1, nl1)
    in_specs += [fspec, fspec]
    h1 = pl.pallas_call(
        functools.partial(_cross_kernel, nl=nl1),
        out_shape=jax.ShapeDtypeStruct((2, R, D), jnp.float32),
        grid=(2, nl1),
        in_specs=in_specs,
        out_specs=pl.BlockSpec((1, R, D), lambda b, l: (b, 0, 0)),
        scratch_shapes=[pltpu.VMEM((R, D), jnp.float32),
                        pltpu.VMEM((R, D), jnp.float32)],
        compiler_params=sem,
    )(q_emb, qw, qb, k_emb, kw, kb, pos, bias, *kq, *qk, f1g, f1b)

    NO = proj2_w.shape[1]
    C = glove_T.shape[1]
    CP = ((C + 127) // 128) * 128
    gT_pad = jnp.pad(glove_T, ((0, 0), (0, CP - C)))
    cand_bias = jnp.concatenate(
        [jnp.zeros((1, C), jnp.float32),
         jnp.full((1, CP - C), -1e30, jnp.float32)], axis=1)

    nl2 = kmem_wq.shape[0]
    in_specs = [
        pl.BlockSpec((1, R, D), lambda b, l: (b, 0, 0)),
        res((R, D)), res((T, T)), res((B, R)),
        res(proj1_w.shape), res(proj1_b.shape),
        res(proj2_w.shape), res(proj2_b.shape),
        res((NO, CP)), res((1, CP)),
    ]
    in_specs += _stack_specs([p.shape for p in kmem], 0, nl2)
    in_specs += _stack_specs([p.shape for p in qmem], 1, nl2)
    in_specs += [fspec, fspec]
    pred = pl.pallas_call(
        functools.partial(_mem_kernel, nl=nl2, scale=scale),
        out_shape=jax.ShapeDtypeStruct((B, CP), jnp.float32),
        grid=(2, nl2),
        in_specs=in_specs,
        out_specs=pl.BlockSpec((B, CP), lambda b, l: (0, 0)),
        scratch_shapes=[pltpu.VMEM((R, D), jnp.float32),
                        pltpu.VMEM((R, D), jnp.float32)],
        compiler_params=sem,
    )(h1, pos, bias, pool, proj1_w, proj1_b, proj2_w, proj2_b,
      gT_pad, cand_bias, *kmem, *qmem, f2g, f2b)
    return pred[:, :C]


# speedup vs baseline: 1.1597x; 1.0032x over previous
import functools
import math

import numpy as np
import jax
import jax.numpy as jnp
from jax import lax
from jax.experimental import pallas as pl
from jax.experimental.pallas import tpu as pltpu

_NH = 8
_F32 = jnp.float32


def _ln(x, g, b, eps=1e-5):
    mu = jnp.mean(x, axis=-1, keepdims=True)
    var = jnp.mean((x - mu) ** 2, axis=-1, keepdims=True)
    return (x - mu) * lax.rsqrt(var + eps) * g + b


def _pos_emb_np(L, D):
    half = D // 2
    freqs = np.exp(np.arange(half) * -(math.log(10000.0) / (half - 1)))
    pos = np.arange(1, L + 1, dtype=np.float32)
    args = pos[:, None] * freqs[None, :]
    pe = np.concatenate([np.sin(args), np.cos(args)], axis=1)
    if D % 2 == 1:
        pe = np.concatenate([pe, np.zeros((L, 1))], axis=1)
    return pe.astype(np.float32)


def _tile_rows(pat, R):
    L = pat.shape[0]
    if L == R:
        return pat
    return jnp.concatenate([pat] * (R // L), axis=0)


def _layer_body(x_scr, kv_scr, bias, p, *, cross):
    ln0g, ln0b, wq, bq, wkv, bkv, wo, bo, ln1g, ln1b, w1, b1, w2, b2 = p
    R, D = x_scr.shape
    dh = D // _NH
    T = bias.shape[0]
    CS = R
    g0 = ln0g[0]
    b0 = ln0b[0]
    xfull = x_scr[...]
    kvfull = kv_scr[...] if cross else None
    xout = []
    for c in range(R // CS):
        rs = slice(c * CS, (c + 1) * CS)
        x = xfull[rs, :]
        xn = _ln(x, g0, b0)
        kvn = _ln(kvfull[rs, :], g0, b0) if cross else xn

        q = jnp.dot(xn, wq[0], preferred_element_type=_F32) + bq[0]
        kvp = jnp.dot(kvn, wkv[0], preferred_element_type=_F32) + bkv[0]
        k = kvp[:, :D]
        v = kvp[:, D:]

        tiles = [(h, g) for h in range(_NH) for g in range(CS // T)]
        ss = {}
        for h, g in tiles:
            sl = slice(g * T, (g + 1) * T)
            cols = slice(h * dh, (h + 1) * dh)
            ss[h, g] = lax.dot_general(q[sl, cols], k[sl, cols],
                                       (((1,), (1,)), ((), ())),
                                       preferred_element_type=_F32)
        es = {}
        for h, g in tiles:
            s = ss[h, g] + bias
            m = jnp.max(s, axis=-1, keepdims=True)
            e = jnp.exp(s - m)
            es[h, g] = (e, 1.0 / jnp.sum(e, axis=-1, keepdims=True))
        heads = []
        for h in range(_NH):
            outs = []
            for g in range(CS // T):
                sl = slice(g * T, (g + 1) * T)
                e, r = es[h, g]
                outs.append(
                    jnp.dot(e, v[sl, h * dh:(h + 1) * dh],
                            preferred_element_type=_F32) * r)
            heads.append(jnp.concatenate(outs, axis=0))
        attn = jnp.concatenate(heads, axis=-1)
        x = x + jnp.dot(attn, wo[0], preferred_element_type=_F32) + bo[0]

        xn2 = _ln(x, ln1g[0], ln1b[0])
        h1 = jnp.maximum(
            jnp.dot(xn2, w1[0], preferred_element_type=_F32) + b1[0], 0.0)
        x = x + jnp.dot(h1, w2[0], preferred_element_type=_F32) + b2[0]
        xout.append((rs, x))
    for rs, x in xout:
        x_scr[rs, :] = x
    return xout


def _cross_kernel(qe_ref, qw_ref, qb_ref, ke_ref, kw_ref, kb_ref,
                  pos_ref, bias_ref, *rest, nl):
    p0 = rest[:14]
    p1 = rest[14:28]
    f1g, f1b = rest[28], rest[29]
    o_ref = rest[30]
    x_scr, kv_scr = rest[31], rest[32]

    b = pl.program_id(0)
    l = pl.program_id(1)

    @pl.when(l == 0)
    def _():
        hq = jnp.dot(qe_ref[...], qw_ref[...],
                     preferred_element_type=_F32) + qb_ref[...]
        hk = jnp.dot(ke_ref[...], kw_ref[...],
                     preferred_element_type=_F32) + kb_ref[...]
        pos = _tile_rows(pos_ref[...], hq.shape[0])
        x_scr[...] = jnp.where(b == 0, hk, hq) + pos
        kv_scr[...] = jnp.where(b == 0, hq, hk) + pos

    def run(p):
        xout = _layer_body(x_scr, kv_scr, bias_ref[...], p, cross=True)

        @pl.when(l == nl - 1)
        def _():
            for rs, x in xout:
                o_ref[0, rs, :] = _ln(x, f1g[0], f1b[0])

    @pl.when(b == 0)
    def _():
        run(p0)

    @pl.when(b == 1)
    def _():
        run(p1)


def _mem_kernel(x_ref, pos_ref, bias_ref, pool_ref, w1_ref, b1_ref,
                w2_ref, b2_ref, gT_ref, cb_ref, *rest, nl, scale):
    p0 = rest[:14]
    p1 = rest[14:28]
    f2g, f2b = rest[28], rest[29]
    o_ref = rest[30]
    x_scr, ks_scr = rest[31], rest[32]

    b = pl.program_id(0)
    l = pl.program_id(1)

    @pl.when(l == 0)
    def _():
        x = scale * x_ref[0]
        x_scr[...] = x + _tile_rows(pos_ref[...], x.shape[0])

    @pl.when(b == 0)
    def _():
        xout = _layer_body(x_scr, None, bias_ref[...], p0, cross=False)

        @pl.when(l == nl - 1)
        def _():
            pool = pool_ref[...]
            ks = None
            for rs, x in xout:
                part = jnp.dot(pool[:, rs], _ln(x, f2g[0], f2b[0]),
                               preferred_element_type=_F32)
                ks = part if ks is None else ks + part
            ks_scr[...] = ks

    @pl.when(b == 1)
    def _():
        xout = _layer_body(x_scr, None, bias_ref[...], p1, cross=False)

        @pl.when(l == nl - 1)
        def _():
            pool = pool_ref[...]
            ks = ks_scr[...]
            qs = None
            for rs, x in xout:
                part = jnp.dot(pool[:, rs], _ln(x, f2g[0], f2b[0]),
                               preferred_element_type=_F32)
                qs = part if qs is None else qs + part
            last = jnp.concatenate([ks, qs], axis=-1)
            h = jnp.maximum(
                jnp.dot(last, w1_ref[...], preferred_element_type=_F32)
                + b1_ref[...], 0.0)
            out = (jnp.dot(h, w2_ref[...], preferred_element_type=_F32)
                   + b2_ref[...])
            sim = (jnp.dot(out, gT_ref[...], preferred_element_type=_F32)
                   + cb_ref[...])
            m = jnp.max(sim, axis=-1, keepdims=True)
            lse = m + jnp.log(jnp.sum(jnp.exp(sim - m), axis=-1,
                                      keepdims=True))
            o_ref[...] = sim - lse


def _stack_specs(shapes, own_branch, nl):
    specs = []
    for shp in shapes:
        if own_branch == 0:
            def idx(b, l, _n=nl):
                return (jnp.where(b == 0, l, _n - 1), 0, 0)
        else:
            def idx(b, l):
                return (jnp.where(b == 1, l, 0), 0, 0)
        specs.append(pl.BlockSpec((1,) + shp[1:], idx))
    return specs


def kernel(he_ques, he_kg, emb, q2h_w, q2h_b, k2h_w, k2h_b,
           kq_ln0g, kq_ln0b, kq_wq, kq_bq, kq_wkv, kq_bkv, kq_wo, kq_bo,
           kq_ln1g, kq_ln1b, kq_w1, kq_b1, kq_w2, kq_b2, kq_fg, kq_fb,
           qk_ln0g, qk_ln0b, qk_wq, qk_bq, qk_wkv, qk_bkv, qk_wo, qk_bo,
           qk_ln1g, qk_ln1b, qk_w1, qk_b1, qk_w2, qk_b2, qk_fg, qk_fb,
           kmem_ln0g, kmem_ln0b, kmem_wq, kmem_bq, kmem_wkv, kmem_bkv,
           kmem_wo, kmem_bo, kmem_ln1g, kmem_ln1b, kmem_w1, kmem_b1,
           kmem_w2, kmem_b2, kmem_fg, kmem_fb,
           qmem_ln0g, qmem_ln0b, qmem_wq, qmem_bq, qmem_wkv, qmem_bkv,
           qmem_wo, qmem_bo, qmem_ln1g, qmem_ln1b, qmem_w1, qmem_b1,
           qmem_w2, qmem_b2, qmem_fg, qmem_fb,
           proj1_w, proj1_b, proj2_w, proj2_b, glove_T):
    B, Lq, _ = he_ques.shape
    _, Lk, _ = he_kg.shape
    D = q2h_w.shape[1]
    R = B * Lq
    assert Lq == Lk, "fused dual-branch layout needs equal slab shapes"
    scale = math.sqrt(D)

    q_emb = emb[he_ques].reshape(R, -1)
    k_emb = emb[he_kg].reshape(R, -1)

    pos = jnp.asarray(_pos_emb_np(Lq, D))
    blk = np.repeat(np.arange(B), Lq)
    pool = jnp.asarray(
        (blk[None, :] == np.arange(B)[:, None]).astype(np.float32))
    T = math.gcd(R, 128)
    if T % Lq != 0:
        T = R
    tb = np.repeat(np.arange(T // Lq), Lq)
    bias = jnp.asarray(
        np.where(tb[:, None] == tb[None, :], 0.0, -1e30).astype(np.float32))

    qw = q2h_w * scale
    qb = q2h_b * scale
    kw = k2h_w * scale
    kb = k2h_b * scale

    kq = (kq_ln0g, kq_ln0b, kq_wq, kq_bq, kq_wkv, kq_bkv, kq_wo, kq_bo,
          kq_ln1g, kq_ln1b, kq_w1, kq_b1, kq_w2, kq_b2)
    qk = (qk_ln0g, qk_ln0b, qk_wq, qk_bq, qk_wkv, qk_bkv, qk_wo, qk_bo,
          qk_ln1g, qk_ln1b, qk_w1, qk_b1, qk_w2, qk_b2)
    kmem = (kmem_ln0g, kmem_ln0b, kmem_wq, kmem_bq, kmem_wkv, kmem_bkv,
            kmem_wo, kmem_bo, kmem_ln1g, kmem_ln1b, kmem_w1, kmem_b1,
            kmem_w2, kmem_b2)
    qmem = (qmem_ln0g, qmem_ln0b, qmem_wq, qmem_bq, qmem_wkv, qmem_bkv,
            qmem_wo, qmem_bo, qmem_ln1g, qmem_ln1b, qmem_w1, qmem_b1,
            qmem_w2, qmem_b2)

    f1g = jnp.stack([kq_fg, qk_fg])
    f1b = jnp.stack([kq_fb, qk_fb])
    f2g = jnp.stack([kmem_fg, qmem_fg])
    f2b = jnp.stack([kmem_fb, qmem_fb])

    sem = pltpu.CompilerParams(
        dimension_semantics=("arbitrary", "arbitrary"),
        vmem_limit_bytes=64 * 1024 * 1024)
    res = lambda shp: pl.BlockSpec(shp, lambda b, l: (0,) * len(shp))
    fspec = pl.BlockSpec((1, 1, D), lambda b, l: (b, 0, 0))

    nl1 = kq_wq.shape[0]
    in_specs = [
        res(q_emb.shape), res(qw.shape), res(qb.shape),
        res(k_emb.shape), res(kw.shape), res(kb.shape),
        res((Lq, D)), res((T, T)),
    ]
    in_specs += _stack_specs([p.shape for p in kq], 0, nl1)
    in_specs += _stack_specs([p.shape for p in qk], 1, nl1)
    in_specs += [fspec, fspec]
    h1 = pl.pallas_call(
        functools.partial(_cross_kernel, nl=nl1),
        out_shape=jax.ShapeDtypeStruct((2, R, D), jnp.float32),
        grid=(2, nl1),
        in_specs=in_specs,
        out_specs=pl.BlockSpec((1, R, D), lambda b, l: (b, 0, 0)),
        scratch_shapes=[pltpu.VMEM((R, D), jnp.float32),
                        pltpu.VMEM((R, D), jnp.float32)],
        compiler_params=sem,
    )(q_emb, qw, qb, k_emb, kw, kb, pos, bias, *kq, *qk, f1g, f1b)

    NO = proj2_w.shape[1]
    C = glove_T.shape[1]
    CP = ((C + 127) // 128) * 128
    gT_pad = jnp.pad(glove_T, ((0, 0), (0, CP - C)))
    cand_bias = jnp.concatenate(
        [jnp.zeros((1, C), jnp.float32),
         jnp.full((1, CP - C), -1e30, jnp.float32)], axis=1)

    nl2 = kmem_wq.shape[0]
    in_specs = [
        pl.BlockSpec((1, R, D), lambda b, l: (b, 0, 0)),
        res((Lq, D)), res((T, T)), res((B, R)),
        res(proj1_w.shape), res(proj1_b.shape),
        res(proj2_w.shape), res(proj2_b.shape),
        res((NO, CP)), res((1, CP)),
    ]
    in_specs += _stack_specs([p.shape for p in kmem], 0, nl2)
    in_specs += _stack_specs([p.shape for p in qmem], 1, nl2)
    in_specs += [fspec, fspec]
    pred = pl.pallas_call(
        functools.partial(_mem_kernel, nl=nl2, scale=scale),
        out_shape=jax.ShapeDtypeStruct((B, CP), jnp.float32),
        grid=(2, nl2),
        in_specs=in_specs,
        out_specs=pl.BlockSpec((B, CP), lambda b, l: (0, 0)),
        scratch_shapes=[pltpu.VMEM((R, D), jnp.float32),
                        pltpu.VMEM((B, D), jnp.float32)],
        compiler_params=sem,
    )(h1, pos, bias, pool, proj1_w, proj1_b, proj2_w, proj2_b,
      gT_pad, cand_bias, *kmem, *qmem, f2g, f2b)
    return pred[:, :C]


# speedup vs baseline: 1.1662x; 1.0056x over previous
import functools
import math

import numpy as np
import jax
import jax.numpy as jnp
from jax import lax
from jax.experimental import pallas as pl
from jax.experimental.pallas import tpu as pltpu

_NH = 8
_F32 = jnp.float32


def _ln(x, g, b, eps=1e-5):
    mu = jnp.mean(x, axis=-1, keepdims=True)
    var = jnp.mean((x - mu) ** 2, axis=-1, keepdims=True)
    return (x - mu) * lax.rsqrt(var + eps) * g + b


def _pos_emb_np(L, D):
    half = D // 2
    freqs = np.exp(np.arange(half) * -(math.log(10000.0) / (half - 1)))
    pos = np.arange(1, L + 1, dtype=np.float32)
    args = pos[:, None] * freqs[None, :]
    pe = np.concatenate([np.sin(args), np.cos(args)], axis=1)
    if D % 2 == 1:
        pe = np.concatenate([pe, np.zeros((L, 1))], axis=1)
    return pe.astype(np.float32)


def _tile_rows(pat, R):
    L = pat.shape[0]
    if L == R:
        return pat
    return jnp.concatenate([pat] * (R // L), axis=0)


def _layer_body(x_scr, kv_scr, bias, p, *, cross):
    ln0g, ln0b, wq, bq, wkv, bkv, wo, bo, ln1g, ln1b, w1, b1, w2, b2 = p
    R, D = x_scr.shape
    dh = D // _NH
    T = bias.shape[0]
    CS = R
    g0 = ln0g[0]
    b0 = ln0b[0]
    xfull = x_scr[...]
    kvfull = kv_scr[...] if cross else None
    xout = []
    for c in range(R // CS):
        rs = slice(c * CS, (c + 1) * CS)
        x = xfull[rs, :]
        xn = _ln(x, g0, b0)
        kvn = _ln(kvfull[rs, :], g0, b0) if cross else xn

        q = jnp.dot(xn, wq[0], preferred_element_type=_F32) + bq[0]
        kvp = jnp.dot(kvn, wkv[0], preferred_element_type=_F32) + bkv[0]
        k = kvp[:, :D]
        v = kvp[:, D:]

        tiles = [(h, g) for h in range(_NH) for g in range(CS // T)]
        ss = {}
        for h, g in tiles:
            sl = slice(g * T, (g + 1) * T)
            cols = slice(h * dh, (h + 1) * dh)
            ss[h, g] = lax.dot_general(q[sl, cols], k[sl, cols],
                                       (((1,), (1,)), ((), ())),
                                       preferred_element_type=_F32)
        es = {}
        for h, g in tiles:
            s = ss[h, g] + bias
            m = jnp.max(s, axis=-1, keepdims=True)
            e = jnp.exp(s - m)
            es[h, g] = (e, 1.0 / jnp.sum(e, axis=-1, keepdims=True))
        heads = []
        for h in range(_NH):
            outs = []
            for g in range(CS // T):
                sl = slice(g * T, (g + 1) * T)
                e, r = es[h, g]
                outs.append(
                    jnp.dot(e, v[sl, h * dh:(h + 1) * dh],
                            preferred_element_type=_F32) * r)
            heads.append(jnp.concatenate(outs, axis=0))
        attn = jnp.concatenate(heads, axis=-1)
        x = x + jnp.dot(attn, wo[0], preferred_element_type=_F32) + bo[0]

        xn2 = _ln(x, ln1g[0], ln1b[0])
        h1 = jnp.maximum(
            jnp.dot(xn2, w1[0], preferred_element_type=_F32) + b1[0], 0.0)
        x = x + jnp.dot(h1, w2[0], preferred_element_type=_F32) + b2[0]
        xout.append((rs, x))
    for rs, x in xout:
        x_scr[rs, :] = x
    return xout


def _cross_kernel(qe_ref, qw_ref, qb_ref, ke_ref, kw_ref, kb_ref,
                  pos_ref, bias_ref, *rest, nl):
    p0 = rest[:14]
    p1 = rest[14:28]
    f1g, f1b = rest[28], rest[29]
    o_ref = rest[30]
    x0_scr, x1_scr, kv0_scr, kv1_scr = rest[31], rest[32], rest[33], rest[34]

    l = pl.program_id(0)

    @pl.when(l == 0)
    def _():
        hq = jnp.dot(qe_ref[...], qw_ref[...],
                     preferred_element_type=_F32) + qb_ref[...]
        hk = jnp.dot(ke_ref[...], kw_ref[...],
                     preferred_element_type=_F32) + kb_ref[...]
        pos = _tile_rows(pos_ref[...], hq.shape[0])
        hq = hq + pos
        hk = hk + pos
        x0_scr[...] = hk
        kv0_scr[...] = hq
        x1_scr[...] = hq
        kv1_scr[...] = hk

    bias = bias_ref[...]
    xout0 = _layer_body(x0_scr, kv0_scr, bias, p0, cross=True)
    xout1 = _layer_body(x1_scr, kv1_scr, bias, p1, cross=True)

    @pl.when(l == nl - 1)
    def _():
        for rs, x in xout0:
            o_ref[0, rs, :] = _ln(x, f1g[0], f1b[0])
        for rs, x in xout1:
            o_ref[1, rs, :] = _ln(x, f1g[1], f1b[1])


def _mem_kernel(x_ref, pos_ref, bias_ref, pool_ref, w1_ref, b1_ref,
                w2_ref, b2_ref, gT_ref, cb_ref, *rest, nl, scale):
    p0 = rest[:14]
    p1 = rest[14:28]
    f2g, f2b = rest[28], rest[29]
    o_ref = rest[30]
    x0_scr, x1_scr = rest[31], rest[32]

    l = pl.program_id(0)

    @pl.when(l == 0)
    def _():
        pos = _tile_rows(pos_ref[...], x0_scr.shape[0])
        x0_scr[...] = scale * x_ref[0] + pos
        x1_scr[...] = scale * x_ref[1] + pos

    bias = bias_ref[...]
    xout0 = _layer_body(x0_scr, None, bias, p0, cross=False)
    xout1 = _layer_body(x1_scr, None, bias, p1, cross=False)

    @pl.when(l == nl - 1)
    def _():
        pool = pool_ref[...]

        def pooled(xout, g, b):
            acc = None
            for rs, x in xout:
                part = jnp.dot(pool[:, rs], _ln(x, g, b),
                               preferred_element_type=_F32)
                acc = part if acc is None else acc + part
            return acc

        ks = pooled(xout0, f2g[0], f2b[0])
        qs = pooled(xout1, f2g[1], f2b[1])
        last = jnp.concatenate([ks, qs], axis=-1)
        h = jnp.maximum(
            jnp.dot(last, w1_ref[...], preferred_element_type=_F32)
            + b1_ref[...], 0.0)
        out = (jnp.dot(h, w2_ref[...], preferred_element_type=_F32)
               + b2_ref[...])
        sim = (jnp.dot(out, gT_ref[...], preferred_element_type=_F32)
               + cb_ref[...])
        m = jnp.max(sim, axis=-1, keepdims=True)
        lse = m + jnp.log(jnp.sum(jnp.exp(sim - m), axis=-1,
                                  keepdims=True))
        o_ref[...] = sim - lse


def _stack_specs(shapes):
    return [pl.BlockSpec((1,) + shp[1:], lambda l: (l, 0, 0))
            for shp in shapes]


def kernel(he_ques, he_kg, emb, q2h_w, q2h_b, k2h_w, k2h_b,
           kq_ln0g, kq_ln0b, kq_wq, kq_bq, kq_wkv, kq_bkv, kq_wo, kq_bo,
           kq_ln1g, kq_ln1b, kq_w1, kq_b1, kq_w2, kq_b2, kq_fg, kq_fb,
           qk_ln0g, qk_ln0b, qk_wq, qk_bq, qk_wkv, qk_bkv, qk_wo, qk_bo,
           qk_ln1g, qk_ln1b, qk_w1, qk_b1, qk_w2, qk_b2, qk_fg, qk_fb,
           kmem_ln0g, kmem_ln0b, kmem_wq, kmem_bq, kmem_wkv, kmem_bkv,
           kmem_wo, kmem_bo, kmem_ln1g, kmem_ln1b, kmem_w1, kmem_b1,
           kmem_w2, kmem_b2, kmem_fg, kmem_fb,
           qmem_ln0g, qmem_ln0b, qmem_wq, qmem_bq, qmem_wkv, qmem_bkv,
           qmem_wo, qmem_bo, qmem_ln1g, qmem_ln1b, qmem_w1, qmem_b1,
           qmem_w2, qmem_b2, qmem_fg, qmem_fb,
           proj1_w, proj1_b, proj2_w, proj2_b, glove_T):
    B, Lq, _ = he_ques.shape
    _, Lk, _ = he_kg.shape
    D = q2h_w.shape[1]
    R = B * Lq
    assert Lq == Lk, "fused dual-branch layout needs equal slab shapes"
    scale = math.sqrt(D)

    q_emb = emb[he_ques].reshape(R, -1)
    k_emb = emb[he_kg].reshape(R, -1)

    pos = jnp.asarray(_pos_emb_np(Lq, D))
    blk = np.repeat(np.arange(B), Lq)
    pool = jnp.asarray(
        (blk[None, :] == np.arange(B)[:, None]).astype(np.float32))
    T = math.gcd(R, 128)
    if T % Lq != 0:
        T = R
    tb = np.repeat(np.arange(T // Lq), Lq)
    bias = jnp.asarray(
        np.where(tb[:, None] == tb[None, :], 0.0, -1e30).astype(np.float32))

    qw = q2h_w * scale
    qb = q2h_b * scale
    kw = k2h_w * scale
    kb = k2h_b * scale

    kq = (kq_ln0g, kq_ln0b, kq_wq, kq_bq, kq_wkv, kq_bkv, kq_wo, kq_bo,
          kq_ln1g, kq_ln1b, kq_w1, kq_b1, kq_w2, kq_b2)
    qk = (qk_ln0g, qk_ln0b, qk_wq, qk_bq, qk_wkv, qk_bkv, qk_wo, qk_bo,
          qk_ln1g, qk_ln1b, qk_w1, qk_b1, qk_w2, qk_b2)
    kmem = (kmem_ln0g, kmem_ln0b, kmem_wq, kmem_bq, kmem_wkv, kmem_bkv,
            kmem_wo, kmem_bo, kmem_ln1g, kmem_ln1b, kmem_w1, kmem_b1,
            kmem_w2, kmem_b2)
    qmem = (qmem_ln0g, qmem_ln0b, qmem_wq, qmem_bq, qmem_wkv, qmem_bkv,
            qmem_wo, qmem_bo, qmem_ln1g, qmem_ln1b, qmem_w1, qmem_b1,
            qmem_w2, qmem_b2)

    f1g = jnp.stack([kq_fg, qk_fg])
    f1b = jnp.stack([kq_fb, qk_fb])
    f2g = jnp.stack([kmem_fg, qmem_fg])
    f2b = jnp.stack([kmem_fb, qmem_fb])

    sem = pltpu.CompilerParams(
        dimension_semantics=("arbitrary",),
        vmem_limit_bytes=64 * 1024 * 1024)
    res = lambda shp: pl.BlockSpec(shp, lambda l: (0,) * len(shp))

    nl1 = kq_wq.shape[0]
    in_specs = [
        res(q_emb.shape), res(qw.shape), res(qb.shape),
        res(k_emb.shape), res(kw.shape), res(kb.shape),
        res((Lq, D)), res((T, T)),
    ]
    in_specs += _stack_specs([p.shape for p in kq])
    in_specs += _stack_specs([p.shape for p in qk])
    in_specs += [res((2, 1, D)), res((2, 1, D))]
    h1 = pl.pallas_call(
        functools.partial(_cross_kernel, nl=nl1),
        out_shape=jax.ShapeDtypeStruct((2, R, D), jnp.float32),
        grid=(nl1,),
        in_specs=in_specs,
        out_specs=res((2, R, D)),
        scratch_shapes=[pltpu.VMEM((R, D), jnp.float32),
                        pltpu.VMEM((R, D), jnp.float32),
                        pltpu.VMEM((R, D), jnp.float32),
                        pltpu.VMEM((R, D), jnp.float32)],
        compiler_params=sem,
    )(q_emb, qw, qb, k_emb, kw, kb, pos, bias, *kq, *qk, f1g, f1b)

    NO = proj2_w.shape[1]
    C = glove_T.shape[1]
    CP = ((C + 127) // 128) * 128
    gT_pad = jnp.pad(glove_T, ((0, 0), (0, CP - C)))
    cand_bias = jnp.concatenate(
        [jnp.zeros((1, C), jnp.float32),
         jnp.full((1, CP - C), -1e30, jnp.float32)], axis=1)

    nl2 = kmem_wq.shape[0]
    in_specs = [
        res((2, R, D)),
        res((Lq, D)), res((T, T)), res((B, R)),
        res(proj1_w.shape), res(proj1_b.shape),
        res(proj2_w.shape), res(proj2_b.shape),
        res((NO, CP)), res((1, CP)),
    ]
    in_specs += _stack_specs([p.shape for p in kmem])
    in_specs += _stack_specs([p.shape for p in qmem])
    in_specs += [res((2, 1, D)), res((2, 1, D))]
    pred = pl.pallas_call(
        functools.partial(_mem_kernel, nl=nl2, scale=scale),
        out_shape=jax.ShapeDtypeStruct((B, CP), jnp.float32),
        grid=(nl2,),
        in_specs=in_specs,
        out_specs=res((B, CP)),
        scratch_shapes=[pltpu.VMEM((R, D), jnp.float32),
                        pltpu.VMEM((R, D), jnp.float32)],
        compiler_params=sem,
    )(h1, pos, bias, pool, proj1_w, proj1_b, proj2_w, proj2_b,
      gT_pad, cand_bias, *kmem, *qmem, f2g, f2b)
    return pred[:, :C]
